# baseline (speedup 1.0000x reference)
import jax
import jax.numpy as jnp
from jax import lax
from jax.experimental import pallas as pl
from jax.experimental.pallas import tpu as pltpu
from jax.experimental.pallas import tpu_sc as plsc

_B, _T, _H, _W = 256, 64, 256, 256
_HW = _H * _W
_NC, _NS = 2, 16
_NW = _NC * _NS
_BPW = _B // _NW
_L = 16


def _static_int(v, default):
    try:
        return int(v)
    except Exception:
        return default


def _build(n_guide, obs_len):
    mesh = plsc.VectorSubcoreMesh(
        core_axis_name="c", subcore_axis_name="s",
        num_cores=_NC, num_subcores=_NS)

    def body(x_hbm, gx_hbm, gy_hbm, cs_hbm, out_hbm,
             xbuf, cenbuf, stdbuf, idxbufa, idxbufb, gxva, gyva, gxvb, gyvb,
             outbuf, sem_xa, sem_ya, sem_xb, sem_yb):
        wid = lax.axis_index("s") * _NC + lax.axis_index("c")
        b0 = wid * _BPW

        pltpu.sync_copy(x_hbm.at[pl.ds(b0, _BPW)], xbuf)
        pltpu.sync_copy(cs_hbm.at[pl.ds(b0 * 2, _BPW * 2)], cenbuf)
        pltpu.sync_copy(cs_hbm.at[pl.ds(_B * 2 + b0 * 2, _BPW * 2)], stdbuf)

        lane = lax.iota(jnp.int32, _L)
        lane8 = jnp.bitwise_and(lane, _BPW - 1)
        act = lane < _BPW
        pair = lane8 * 2
        c0 = plsc.load_gather(cenbuf, [pair])
        c1 = plsc.load_gather(cenbuf, [pair + 1])
        s0 = plsc.load_gather(stdbuf, [pair])
        s1 = plsc.load_gather(stdbuf, [pair + 1])
        base_row = (b0 + lane8) * _H

        zeros = jnp.zeros((_L,), jnp.float32)
        for i in range(_BPW):
            for j in range(_T * 2 // _L):
                outbuf[i, pl.ds(j * _L, _L)] = zeros

        idx8a = idxbufa.at[pl.ds(0, _BPW)]
        idx8b = idxbufb.at[pl.ds(0, _BPW)]

        def lc_load(xi, s, c):
            return plsc.load_gather(xbuf, [lane8, xi]) * s + c

        def issue(cl0v, idxb, idx8, bufx, bufy, semx, semy):
            idxb[...] = base_row + cl0v
            pltpu.async_copy(gx_hbm.at[idx8], bufx, semx)
            pltpu.async_copy(gy_hbm.at[idx8], bufy, semy)

        def wait(bufx, semx):
            pltpu.make_async_copy(gx_hbm.at[idx8a], bufx, semx).wait()

        def clip_f(v):
            return jnp.clip(v, 0.0, float(_H - 1)).astype(jnp.int32)

        def clip_i(v):
            return jnp.clip(v, 0, _H - 1)

        if n_guide == 2:
            xi0 = jnp.full((_L,), obs_len * 2, jnp.int32)
            cl0a = clip_f(lc_load(xi0, s0, c0))
            issue(cl0a, idxbufa, idx8a, gxva, gyva, sem_xa, sem_ya)

            def step(tt, carry):
                cum0, cum1, cl0a = carry
                xi = jnp.full((_L,), tt * 2, jnp.int32)
                cl1a = clip_f(lc_load(xi + 1, s1, c1) + cum1)
                wait(gxva, sem_xa)
                g0a = plsc.load_gather(gxva, [lane8, cl1a])
                cl0b = clip_i(cl0a - g0a.astype(jnp.int32))
                issue(cl0b, idxbufb, idx8b, gxvb, gyvb, sem_xb, sem_yb)
                wait(gyva, sem_ya)
                g1a = plsc.load_gather(gyva, [lane8, cl1a])
                cl1b = clip_i(cl1a - g1a.astype(jnp.int32))
                xin = jnp.minimum(xi + 2, _T * 2 - 2)
                lc0n = lc_load(xin, s0, c0)
                wait(gxvb, sem_xb)
                g0b = plsc.load_gather(gxvb, [lane8, cl1b])
                cum0n = cum0 - g0a - g0b
                cl0an = clip_f(lc0n + cum0n)
                issue(cl0an, idxbufa, idx8a, gxva, gyva, sem_xa, sem_ya)
                wait(gyvb, sem_yb)
                g1b = plsc.load_gather(gyvb, [lane8, cl1b])
                cum1n = cum1 - g1a - g1b
                plsc.store_scatter(outbuf, [lane8, xi], cum0n / s0, mask=act)
                plsc.store_scatter(outbuf, [lane8, xi + 1], cum1n / s1,
                                   mask=act)
                return (cum0n, cum1n, cl0an)

            lax.fori_loop(obs_len, _T, step, (zeros, zeros, cl0a), unroll=2)
            wait(gxva, sem_xa)
            wait(gyva, sem_ya)
        else:
            def step(tt, carry):
                cum0, cum1 = carry
                xi = jnp.full((_L,), tt * 2, jnp.int32)
                cl0 = clip_f(lc_load(xi, s0, c0) + cum0)
                cl1 = clip_f(lc_load(xi + 1, s1, c1) + cum1)
                g0acc = zeros
                g1acc = zeros
                for _ in range(n_guide):
                    issue(cl0, idxbufa, idx8a, gxva, gyva, sem_xa, sem_ya)
                    wait(gxva, sem_xa)
                    wait(gyva, sem_ya)
                    g0 = plsc.load_gather(gxva, [lane8, cl1])
                    g1 = plsc.load_gather(gyva, [lane8, cl1])
                    g0acc = g0acc - g0
                    g1acc = g1acc - g1
                    cl0 = clip_i(cl0 - g0.astype(jnp.int32))
                    cl1 = clip_i(cl1 - g1.astype(jnp.int32))
                cum0 = cum0 + g0acc
                cum1 = cum1 + g1acc
                plsc.store_scatter(outbuf, [lane8, xi], cum0 / s0, mask=act)
                plsc.store_scatter(outbuf, [lane8, xi + 1], cum1 / s1,
                                   mask=act)
                return (cum0, cum1)

            lax.fori_loop(obs_len, _T, step, (zeros, zeros))

        pltpu.sync_copy(outbuf, out_hbm.at[pl.ds(b0, _BPW)])

    return pl.kernel(
        body,
        out_type=jax.ShapeDtypeStruct((_B, _T * 2), jnp.float32),
        mesh=mesh,
        compiler_params=pltpu.CompilerParams(
            needs_layout_passes=False, disable_bounds_checks=True),
        scratch_types=[
            pltpu.VMEM((_BPW, _T * 2), jnp.float32),
            pltpu.VMEM((_L,), jnp.float32),
            pltpu.VMEM((_L,), jnp.float32),
            pltpu.VMEM((_L,), jnp.int32),
            pltpu.VMEM((_L,), jnp.int32),
            pltpu.VMEM((_BPW, _W), jnp.float32),
            pltpu.VMEM((_BPW, _W), jnp.float32),
            pltpu.VMEM((_BPW, _W), jnp.float32),
            pltpu.VMEM((_BPW, _W), jnp.float32),
            pltpu.VMEM((_BPW, _T * 2), jnp.float32),
            pltpu.SemaphoreType.DMA,
            pltpu.SemaphoreType.DMA,
            pltpu.SemaphoreType.DMA,
            pltpu.SemaphoreType.DMA,
        ],
    )


def kernel(x, cond, grad_x, grad_y, center, std_scale, t, n_guide_steps, obs_len):
    ng = _static_int(n_guide_steps, 2)
    ob = max(0, min(_static_int(obs_len, 8), _T))
    if ob == 0:
        x = x.at[:, 0, :].set(cond)
    cs = jnp.concatenate([center.reshape(-1), std_scale.reshape(-1)])
    out2d = _build(ng, ob)(
        x.reshape(_B, _T * 2), grad_x.reshape(_B * _H, _W),
        grad_y.reshape(_B * _H, _W), cs)
    return (jnp.asarray(0), out2d.reshape(_B, _T, 2))


# speedup vs baseline: 1.0022x; 1.0022x over previous
import jax
import jax.numpy as jnp
from jax import lax
from jax.experimental import pallas as pl
from jax.experimental.pallas import tpu as pltpu
from jax.experimental.pallas import tpu_sc as plsc

_B, _T, _H, _W = 256, 64, 256, 256
_HW = _H * _W
_NC, _NS = 2, 16
_NW = _NC * _NS
_BPW = _B // _NW
_L = 16


def _static_int(v, default):
    try:
        return int(v)
    except Exception:
        return default


def _build(n_guide, obs_len):
    mesh = plsc.VectorSubcoreMesh(
        core_axis_name="c", subcore_axis_name="s",
        num_cores=_NC, num_subcores=_NS)

    def body(x_hbm, gx_hbm, gy_hbm, cs_hbm, out_hbm,
             xbuf, cenbuf, stdbuf, idxbufa, idxbufb, gxva, gyva, gxvb, gyvb,
             outbuf, sem_xa, sem_ya, sem_xb, sem_yb):
        wid = lax.axis_index("s") * _NC + lax.axis_index("c")
        b0 = wid * _BPW

        pltpu.sync_copy(x_hbm.at[pl.ds(b0, _BPW)], xbuf)
        pltpu.sync_copy(cs_hbm.at[pl.ds(b0 * 2, _BPW * 2)], cenbuf)
        pltpu.sync_copy(cs_hbm.at[pl.ds(_B * 2 + b0 * 2, _BPW * 2)], stdbuf)

        lane = lax.iota(jnp.int32, _L)
        lane8 = jnp.bitwise_and(lane, _BPW - 1)
        act = lane < _BPW
        pair = lane8 * 2
        c0 = plsc.load_gather(cenbuf, [pair])
        c1 = plsc.load_gather(cenbuf, [pair + 1])
        s0 = plsc.load_gather(stdbuf, [pair])
        s1 = plsc.load_gather(stdbuf, [pair + 1])
        base_row = (b0 + lane8) * _H

        zeros = jnp.zeros((_L,), jnp.float32)
        for i in range(_BPW):
            for j in range(_T * 2 // _L):
                outbuf[i, pl.ds(j * _L, _L)] = zeros

        idx8a = idxbufa.at[pl.ds(0, _BPW)]
        idx8b = idxbufb.at[pl.ds(0, _BPW)]

        def lc_load(xi, s, c):
            return plsc.load_gather(xbuf, [lane8, xi]) * s + c

        def issue(cl0v, idxb, idx8, bufx, bufy, semx, semy):
            idxb[...] = base_row + cl0v
            pltpu.async_copy(gx_hbm.at[idx8], bufx, semx)
            pltpu.async_copy(gy_hbm.at[idx8], bufy, semy)

        def wait(bufx, semx):
            pltpu.make_async_copy(gx_hbm.at[idx8a], bufx, semx).wait()

        def clip_f(v):
            return jnp.clip(v, 0.0, float(_H - 1)).astype(jnp.int32)

        def clip_i(v):
            return jnp.clip(v, 0, _H - 1)

        if n_guide == 2:
            xi0 = jnp.full((_L,), obs_len * 2, jnp.int32)
            cl0a = clip_f(lc_load(xi0, s0, c0))
            issue(cl0a, idxbufa, idx8a, gxva, gyva, sem_xa, sem_ya)

            def step(tt, carry):
                cum0, cum1, cl0a = carry
                xi = jnp.full((_L,), tt * 2, jnp.int32)
                cl1a = clip_f(lc_load(xi + 1, s1, c1) + cum1)
                wait(gxva, sem_xa)
                g0a = plsc.load_gather(gxva, [lane8, cl1a])
                cl0b = clip_i(cl0a - g0a.astype(jnp.int32))
                issue(cl0b, idxbufb, idx8b, gxvb, gyvb, sem_xb, sem_yb)
                wait(gyva, sem_ya)
                g1a = plsc.load_gather(gyva, [lane8, cl1a])
                cl1b = clip_i(cl1a - g1a.astype(jnp.int32))
                xin = jnp.minimum(xi + 2, _T * 2 - 2)
                lc0n = lc_load(xin, s0, c0)
                wait(gxvb, sem_xb)
                g0b = plsc.load_gather(gxvb, [lane8, cl1b])
                cum0n = cum0 - g0a - g0b
                cl0an = clip_f(lc0n + cum0n)
                issue(cl0an, idxbufa, idx8a, gxva, gyva, sem_xa, sem_ya)
                wait(gyvb, sem_yb)
                g1b = plsc.load_gather(gyvb, [lane8, cl1b])
                cum1n = cum1 - g1a - g1b
                plsc.store_scatter(outbuf, [lane8, xi], cum0n / s0, mask=act)
                plsc.store_scatter(outbuf, [lane8, xi + 1], cum1n / s1,
                                   mask=act)
                return (cum0n, cum1n, cl0an)

            lax.fori_loop(obs_len, _T, step, (zeros, zeros, cl0a))
            wait(gxva, sem_xa)
            wait(gyva, sem_ya)
        else:
            def step(tt, carry):
                cum0, cum1 = carry
                xi = jnp.full((_L,), tt * 2, jnp.int32)
                cl0 = clip_f(lc_load(xi, s0, c0) + cum0)
                cl1 = clip_f(lc_load(xi + 1, s1, c1) + cum1)
                g0acc = zeros
                g1acc = zeros
                for _ in range(n_guide):
                    issue(cl0, idxbufa, idx8a, gxva, gyva, sem_xa, sem_ya)
                    wait(gxva, sem_xa)
                    wait(gyva, sem_ya)
                    g0 = plsc.load_gather(gxva, [lane8, cl1])
                    g1 = plsc.load_gather(gyva, [lane8, cl1])
                    g0acc = g0acc - g0
                    g1acc = g1acc - g1
                    cl0 = clip_i(cl0 - g0.astype(jnp.int32))
                    cl1 = clip_i(cl1 - g1.astype(jnp.int32))
                cum0 = cum0 + g0acc
                cum1 = cum1 + g1acc
                plsc.store_scatter(outbuf, [lane8, xi], cum0 / s0, mask=act)
                plsc.store_scatter(outbuf, [lane8, xi + 1], cum1 / s1,
                                   mask=act)
                return (cum0, cum1)

            lax.fori_loop(obs_len, _T, step, (zeros, zeros))

        pltpu.sync_copy(outbuf, out_hbm.at[pl.ds(b0, _BPW)])

    return pl.kernel(
        body,
        out_type=jax.ShapeDtypeStruct((_B, _T * 2), jnp.float32),
        mesh=mesh,
        compiler_params=pltpu.CompilerParams(
            needs_layout_passes=False, disable_bounds_checks=True),
        scratch_types=[
            pltpu.VMEM((_BPW, _T * 2), jnp.float32),
            pltpu.VMEM((_L,), jnp.float32),
            pltpu.VMEM((_L,), jnp.float32),
            pltpu.VMEM((_L,), jnp.int32),
            pltpu.VMEM((_L,), jnp.int32),
            pltpu.VMEM((_BPW, _W), jnp.float32),
            pltpu.VMEM((_BPW, _W), jnp.float32),
            pltpu.VMEM((_BPW, _W), jnp.float32),
            pltpu.VMEM((_BPW, _W), jnp.float32),
            pltpu.VMEM((_BPW, _T * 2), jnp.float32),
            pltpu.SemaphoreType.DMA,
            pltpu.SemaphoreType.DMA,
            pltpu.SemaphoreType.DMA,
            pltpu.SemaphoreType.DMA,
        ],
    )


def kernel(x, cond, grad_x, grad_y, center, std_scale, t, n_guide_steps, obs_len):
    ng = _static_int(n_guide_steps, 2)
    ob = max(0, min(_static_int(obs_len, 8), _T))
    if ob == 0:
        x = x.at[:, 0, :].set(cond)
    cs = jnp.concatenate([center.reshape(-1), std_scale.reshape(-1)])
    out2d = _build(ng, ob)(
        x.reshape(_B, _T * 2), grad_x.reshape(_B * _H, _W),
        grad_y.reshape(_B * _H, _W), cs)
    return (jnp.asarray(0), out2d.reshape(_B, _T, 2))
